# baseline (speedup 1.0000x reference)
import functools

import jax
import jax.numpy as jnp
from jax import lax
from jax.experimental import pallas as pl
from jax.experimental.pallas import tpu as pltpu
from jax.experimental.pallas import tpu_sc as plsc

_N = 3_200_000
_S = 100_000
_NC = 2
_NS = 16
_BLK = 10_000
_TAIL = 128
_IMAX = jnp.iinfo(jnp.int32).max


def _take(v, idx):
    return jnp.take_along_axis(v, idx, axis=0)


def _body(x_hbm, i_hbm, out_hbm, xb, ib, xtb, itb, ovm, iidx,
          e_prev, e_first, e_last, e_next, *, n, s, c, blk, tail):
    iota = lax.iota(jnp.int32, 16)
    nw = _NC * _NS
    wid = lax.axis_index("s") * _NC + lax.axis_index("c")
    base = wid * c
    neg_inf = jnp.float32(-jnp.inf)

    def splat(v, lane):
        return _take(v, jnp.full((16,), lane, jnp.int32))

    def al(v):
        return pl.multiple_of(v, 8)

    pltpu.sync_copy(i_hbm.at[pl.ds(al(jnp.maximum(base - 16, 0)), 16)], e_prev)
    pltpu.sync_copy(i_hbm.at[pl.ds(al(base), 16)], e_first)
    pltpu.sync_copy(i_hbm.at[pl.ds(al(base + c - 16), 16)], e_last)
    pltpu.sync_copy(
        i_hbm.at[pl.ds(al(jnp.minimum(base + c, n - 16)), 16)], e_next)
    prev_v = splat(e_prev[...], 15)
    first_v = splat(e_first[...], 0)
    last_v = splat(e_last[...], 15)
    next_v = splat(e_next[...], 0)

    is0 = wid == 0
    islast = wid == nw - 1
    nf_lo_v = jnp.where(is0, 0, first_v + (prev_v == first_v).astype(jnp.int32))
    nf_hi_v = jnp.where(islast, s, next_v + (last_v == next_v).astype(jnp.int32))
    nf_lo = jnp.max(nf_lo_v)
    nf_hi = jnp.max(nf_hi_v)
    al_lo = (nf_lo // 8) * 8
    cs = jnp.where(is0, -1, prev_v)
    cm = jnp.full((16,), neg_inf, jnp.float32)
    cp = jnp.full((16,), _IMAX, jnp.int32)
    cr = jnp.full((16,), -1, jnp.int32)

    nbi = (nf_hi - al_lo + 127) // 128
    fill = jnp.full((16,), _IMAX, jnp.int32)

    def init_body(b, _):
        o0 = al(al_lo + b * 128)
        for j in range(8):
            ovm[pl.ds(al(o0 + j * 16), 16)] = fill
        return 0
    lax.fori_loop(0, nbi, init_body, 0)

    sh1 = jnp.maximum(iota - 1, 0)
    shl = jnp.minimum(iota + 1, 15)
    lane15 = jnp.full((16,), 15, jnp.int32)
    lane0m = iota == 0

    def step(x_v, i_v, pos0, cs, cm, cp, cr):
        p_v = pos0 + iota
        st = i_v != _take(i_v, sh1)
        rseed = jnp.where(st | lane0m, p_v, -1)
        r = plsc.cummax(rseed)
        m, p = x_v, p_v
        for k in (1, 2, 4, 8):
            idxk = jnp.maximum(iota - k, 0)
            i_sh = _take(i_v, idxk)
            m_sh = _take(m, idxk)
            p_sh = _take(p, idxk)
            tk = (i_sh == i_v) & (m_sh >= m)
            m = jnp.where(tk, m_sh, m)
            p = jnp.where(tk, p_sh, p)
        fr = i_v == cs
        em0 = lane0m & jnp.logical_not(fr) & (cr >= 0)
        tc = fr & (cm >= m)
        m = jnp.where(tc, cm, m)
        p = jnp.where(tc, cp, p)
        r = jnp.where(fr, cr, r)
        eor = i_v != _take(i_v, shl)
        em = eor & (r >= 0)
        plsc.store_scatter(ovm, [i_v], p - r, mask=em)
        plsc.store_scatter(ovm, [cs], cp - cr, mask=em0)
        return (_take(i_v, lane15), _take(m, lane15), _take(p, lane15),
                _take(r, lane15))

    def outer_body(b, carry):
        pltpu.sync_copy(x_hbm.at[pl.ds(al(base + b * blk), blk)], xb)
        pltpu.sync_copy(i_hbm.at[pl.ds(al(base + b * blk), blk)], ib)

        def block_body(t, carry):
            cs, cm, cp, cr = carry
            off = t * 16
            x_v = xb[pl.ds(off, 16)]
            i_v = ib[pl.ds(off, 16)]
            return step(x_v, i_v, base + b * blk + off, cs, cm, cp, cr)
        return lax.fori_loop(0, blk // 16, block_body, carry)

    cs, cm, cp, cr = lax.fori_loop(0, c // blk, outer_body, (cs, cm, cp, cr))

    cs_s = jnp.max(cs)
    cr_s = jnp.max(cr)
    cm_s = jnp.max(cm)
    cp_s = jnp.max(cp)
    pos0 = base + c
    owned = cr_s >= 0

    def tail_cond(state):
        pos, closed, _, _ = state
        return jnp.logical_not(closed) & (pos < n)

    def tail_body(state):
        pos, closed, cm_s, cp_s = state
        rpos = jnp.minimum(pos, n - tail)
        pltpu.sync_copy(x_hbm.at[pl.ds(al(rpos), tail)], xtb)
        pltpu.sync_copy(i_hbm.at[pl.ds(al(rpos), tail)], itb)
        for j in range(tail // 16):
            x_v = xtb[pl.ds(j * 16, 16)]
            i_v = itb[pl.ds(j * 16, 16)]
            p_v = rpos + j * 16 + iota
            mism = i_v != cs_s
            pm = (jnp.cumsum(mism.astype(jnp.int32)) == 0) & jnp.logical_not(closed)
            vm = jnp.max(jnp.where(pm, x_v, neg_inf))
            vp = jnp.min(jnp.where(pm & (x_v == vm), p_v, _IMAX))
            better = vm > cm_s
            nonempty = jnp.any(pm)
            cm_s = jnp.where(nonempty & better, vm, cm_s)
            cp_s = jnp.where(nonempty & better, vp, cp_s)
            closed = closed | jnp.any(mism)
        return rpos + tail, closed, cm_s, cp_s

    _, _, cm_s, cp_s = lax.while_loop(
        tail_cond, tail_body,
        (pos0, jnp.logical_not(owned) | (pos0 >= n), cm_s, cp_s))

    plsc.store_scatter(ovm, [jnp.broadcast_to(cs_s, (16,))],
                       jnp.broadcast_to(cp_s - cr_s, (16,)),
                       mask=lane0m & owned)

    nbo = (nf_hi - al_lo + 127) // 128

    def copy_body(b, _):
        o0 = al(al_lo + b * 128)
        for j in range(8):
            iv = o0 + j * 16 + iota
            iv = jnp.where((iv >= nf_lo) & (iv < nf_hi), iv, s)
            iidx[pl.ds(j * 16, 16)] = iv
        pltpu.sync_copy(ovm.at[pl.ds(o0, 128)], out_hbm.at[iidx])
        return 0
    lax.fori_loop(0, nbo, copy_body, 0)


def _make(n, s, c, blk, tail):
    mesh = plsc.VectorSubcoreMesh(
        core_axis_name="c", subcore_axis_name="s",
        num_cores=_NC, num_subcores=_NS)
    body = functools.partial(_body, n=n, s=s, c=c, blk=blk, tail=tail)
    return pl.kernel(
        body,
        out_type=jax.ShapeDtypeStruct((s + 128,), jnp.int32),
        mesh=mesh,
        compiler_params=pltpu.CompilerParams(needs_layout_passes=False),
        scratch_types=[
            pltpu.VMEM((blk,), jnp.float32),
            pltpu.VMEM((blk,), jnp.int32),
            pltpu.VMEM((tail,), jnp.float32),
            pltpu.VMEM((tail,), jnp.int32),
            pltpu.VMEM((s + 128,), jnp.int32),
            pltpu.VMEM((128,), jnp.int32),
            pltpu.VMEM((16,), jnp.int32),
            pltpu.VMEM((16,), jnp.int32),
            pltpu.VMEM((16,), jnp.int32),
            pltpu.VMEM((16,), jnp.int32),
        ],
    )


def kernel(x, index):
    out = _make(_N, _S, _N // (_NC * _NS), _BLK, _TAIL)(x, index)
    return out[:_S]


# speedup vs baseline: 1.2363x; 1.2363x over previous
import functools

import jax
import jax.numpy as jnp
from jax import lax
from jax.experimental import pallas as pl
from jax.experimental.pallas import tpu as pltpu
from jax.experimental.pallas import tpu_sc as plsc

_N = 3_200_000
_S = 100_000
_NC = 2
_NS = 16
_BLK = 10_000
_TAIL = 128
_IMAX = jnp.iinfo(jnp.int32).max


def _take(v, idx):
    return jnp.take_along_axis(v, idx, axis=0)


def _body(x_hbm, i_hbm, out_hbm, xb, ib, xtb, itb, ovm, iidx,
          e_prev, e_first, e_last, e_next, *, n, s, c, blk, tail):
    iota = lax.iota(jnp.int32, 16)
    nw = _NC * _NS
    wid = lax.axis_index("s") * _NC + lax.axis_index("c")
    base = wid * c
    neg_inf = jnp.float32(-jnp.inf)

    def splat(v, lane):
        return _take(v, jnp.full((16,), lane, jnp.int32))

    def al(v):
        return pl.multiple_of(v, 8)

    pltpu.sync_copy(i_hbm.at[pl.ds(al(jnp.maximum(base - 16, 0)), 16)], e_prev)
    pltpu.sync_copy(i_hbm.at[pl.ds(al(base), 16)], e_first)
    pltpu.sync_copy(i_hbm.at[pl.ds(al(base + c - 16), 16)], e_last)
    pltpu.sync_copy(
        i_hbm.at[pl.ds(al(jnp.minimum(base + c, n - 16)), 16)], e_next)
    prev_v = splat(e_prev[...], 15)
    first_v = splat(e_first[...], 0)
    last_v = splat(e_last[...], 15)
    next_v = splat(e_next[...], 0)

    is0 = wid == 0
    islast = wid == nw - 1
    nf_lo_v = jnp.where(is0, 0, first_v + (prev_v == first_v).astype(jnp.int32))
    nf_hi_v = jnp.where(islast, s, next_v + (last_v == next_v).astype(jnp.int32))
    nf_lo = jnp.max(nf_lo_v)
    nf_hi = jnp.max(nf_hi_v)
    al_lo = (nf_lo // 8) * 8
    cs = jnp.where(is0, -1, prev_v)
    cm = jnp.full((16,), neg_inf, jnp.float32)
    cp = jnp.full((16,), _IMAX, jnp.int32)
    cr = jnp.full((16,), -1, jnp.int32)

    nbi = (nf_hi - al_lo + 127) // 128
    fill = jnp.full((16,), _IMAX, jnp.int32)

    def init_body(b, _):
        o0 = al(al_lo + b * 128)
        for j in range(8):
            ovm[pl.ds(al(o0 + j * 16), 16)] = fill
        return 0
    lax.fori_loop(0, nbi, init_body, 0)

    sh1 = jnp.maximum(iota - 1, 0)
    lane15 = jnp.full((16,), 15, jnp.int32)
    lane0m = iota == 0

    def step(x_v, i_v, pos0, cs, cm, cp, cr):
        p_v = pos0 + iota
        st = i_v != _take(i_v, sh1)
        rseed = jnp.where(st | lane0m, p_v, -1)
        r = plsc.cummax(rseed)
        m, p = x_v, p_v
        for k in (1, 2, 4, 8):
            idxk = jnp.maximum(iota - k, 0)
            m_sh = _take(m, idxk)
            p_sh = _take(p, idxk)
            same = (jnp.logical_not(st) if k == 1
                    else _take(i_v, idxk) == i_v)
            tk = same & (m_sh >= m)
            m = jnp.where(tk, m_sh, m)
            p = jnp.where(tk, p_sh, p)
        fr = i_v == cs
        tc = fr & (cm >= m)
        m = jnp.where(tc, cm, m)
        p = jnp.where(tc, cp, p)
        r = jnp.where(fr, cr, r)
        plsc.store_scatter(ovm, [i_v], p - r, mask=r >= 0)
        return (_take(i_v, lane15), _take(m, lane15), _take(p, lane15),
                _take(r, lane15))

    def outer_body(b, carry):
        pltpu.sync_copy(x_hbm.at[pl.ds(al(base + b * blk), blk)], xb)
        pltpu.sync_copy(i_hbm.at[pl.ds(al(base + b * blk), blk)], ib)

        def block_body(t, carry):
            cs, cm, cp, cr = carry
            off = t * 16
            x_v = xb[pl.ds(off, 16)]
            i_v = ib[pl.ds(off, 16)]
            return step(x_v, i_v, base + b * blk + off, cs, cm, cp, cr)
        return lax.fori_loop(0, blk // 16, block_body, carry)

    cs, cm, cp, cr = lax.fori_loop(0, c // blk, outer_body, (cs, cm, cp, cr))

    cs_s = jnp.max(cs)
    cr_s = jnp.max(cr)
    cm_s = jnp.max(cm)
    cp_s = jnp.max(cp)
    pos0 = base + c
    owned = cr_s >= 0

    def tail_cond(state):
        pos, closed, _, _ = state
        return jnp.logical_not(closed) & (pos < n)

    def tail_body(state):
        pos, closed, cm_s, cp_s = state
        rpos = jnp.minimum(pos, n - tail)
        pltpu.sync_copy(x_hbm.at[pl.ds(al(rpos), tail)], xtb)
        pltpu.sync_copy(i_hbm.at[pl.ds(al(rpos), tail)], itb)
        for j in range(tail // 16):
            x_v = xtb[pl.ds(j * 16, 16)]
            i_v = itb[pl.ds(j * 16, 16)]
            p_v = rpos + j * 16 + iota
            mism = i_v != cs_s
            pm = (jnp.cumsum(mism.astype(jnp.int32)) == 0) & jnp.logical_not(closed)
            vm = jnp.max(jnp.where(pm, x_v, neg_inf))
            vp = jnp.min(jnp.where(pm & (x_v == vm), p_v, _IMAX))
            better = vm > cm_s
            nonempty = jnp.any(pm)
            cm_s = jnp.where(nonempty & better, vm, cm_s)
            cp_s = jnp.where(nonempty & better, vp, cp_s)
            closed = closed | jnp.any(mism)
        return rpos + tail, closed, cm_s, cp_s

    _, _, cm_s, cp_s = lax.while_loop(
        tail_cond, tail_body,
        (pos0, jnp.logical_not(owned) | (pos0 >= n), cm_s, cp_s))

    plsc.store_scatter(ovm, [jnp.broadcast_to(cs_s, (16,))],
                       jnp.broadcast_to(cp_s - cr_s, (16,)),
                       mask=lane0m & owned)

    nbo = (nf_hi - al_lo + 127) // 128

    def copy_body(b, _):
        o0 = al(al_lo + b * 128)
        for j in range(8):
            iv = o0 + j * 16 + iota
            iv = jnp.where((iv >= nf_lo) & (iv < nf_hi), iv, s)
            iidx[pl.ds(j * 16, 16)] = iv
        pltpu.sync_copy(ovm.at[pl.ds(o0, 128)], out_hbm.at[iidx])
        return 0
    lax.fori_loop(0, nbo, copy_body, 0)


def _make(n, s, c, blk, tail):
    mesh = plsc.VectorSubcoreMesh(
        core_axis_name="c", subcore_axis_name="s",
        num_cores=_NC, num_subcores=_NS)
    body = functools.partial(_body, n=n, s=s, c=c, blk=blk, tail=tail)
    return pl.kernel(
        body,
        out_type=jax.ShapeDtypeStruct((s + 128,), jnp.int32),
        mesh=mesh,
        compiler_params=pltpu.CompilerParams(needs_layout_passes=False),
        scratch_types=[
            pltpu.VMEM((blk,), jnp.float32),
            pltpu.VMEM((blk,), jnp.int32),
            pltpu.VMEM((tail,), jnp.float32),
            pltpu.VMEM((tail,), jnp.int32),
            pltpu.VMEM((s + 128,), jnp.int32),
            pltpu.VMEM((128,), jnp.int32),
            pltpu.VMEM((16,), jnp.int32),
            pltpu.VMEM((16,), jnp.int32),
            pltpu.VMEM((16,), jnp.int32),
            pltpu.VMEM((16,), jnp.int32),
        ],
    )


def kernel(x, index):
    out = _make(_N, _S, _N // (_NC * _NS), _BLK, _TAIL)(x, index)
    return out[:_S]


# speedup vs baseline: 1.2409x; 1.0037x over previous
import functools

import jax
import jax.numpy as jnp
from jax import lax
from jax.experimental import pallas as pl
from jax.experimental.pallas import tpu as pltpu
from jax.experimental.pallas import tpu_sc as plsc

_N = 3_200_000
_S = 100_000
_NC = 2
_NS = 16
_BLK = 10_000
_TAIL = 128
_IMAX = jnp.iinfo(jnp.int32).max


def _take(v, idx):
    return jnp.take_along_axis(v, idx, axis=0)


def _body(x_hbm, i_hbm, out_hbm, xb, ib, xtb, itb, ovm, iidx,
          e_prev, e_first, e_last, e_next, *, n, s, c, blk, tail):
    iota = lax.iota(jnp.int32, 16)
    nw = _NC * _NS
    wid = lax.axis_index("s") * _NC + lax.axis_index("c")
    base = wid * c
    neg_inf = jnp.float32(-jnp.inf)

    def splat(v, lane):
        return _take(v, jnp.full((16,), lane, jnp.int32))

    def al(v):
        return pl.multiple_of(v, 8)

    pltpu.sync_copy(i_hbm.at[pl.ds(al(jnp.maximum(base - 16, 0)), 16)], e_prev)
    pltpu.sync_copy(i_hbm.at[pl.ds(al(base), 16)], e_first)
    pltpu.sync_copy(i_hbm.at[pl.ds(al(base + c - 16), 16)], e_last)
    pltpu.sync_copy(
        i_hbm.at[pl.ds(al(jnp.minimum(base + c, n - 16)), 16)], e_next)
    prev_v = splat(e_prev[...], 15)
    first_v = splat(e_first[...], 0)
    last_v = splat(e_last[...], 15)
    next_v = splat(e_next[...], 0)

    is0 = wid == 0
    islast = wid == nw - 1
    nf_lo_v = jnp.where(is0, 0, first_v + (prev_v == first_v).astype(jnp.int32))
    nf_hi_v = jnp.where(islast, s, next_v + (last_v == next_v).astype(jnp.int32))
    nf_lo = jnp.max(nf_lo_v)
    nf_hi = jnp.max(nf_hi_v)
    al_lo = (nf_lo // 8) * 8
    cs = jnp.where(is0, -1, prev_v)
    cm = jnp.full((16,), neg_inf, jnp.float32)
    cp = jnp.full((16,), _IMAX, jnp.int32)
    cr = jnp.full((16,), -1, jnp.int32)

    nbi = (nf_hi - al_lo + 127) // 128
    fill = jnp.full((16,), _IMAX, jnp.int32)

    def init_body(b, _):
        o0 = al(al_lo + b * 128)
        for j in range(8):
            ovm[pl.ds(al(o0 + j * 16), 16)] = fill
        return 0
    lax.fori_loop(0, nbi, init_body, 0)

    sh1 = jnp.maximum(iota - 1, 0)
    lane15 = jnp.full((16,), 15, jnp.int32)
    lane0m = iota == 0

    def step(x_v, i_v, pos0, cs, cm, cp, cr):
        p_v = pos0 + iota
        st = i_v != _take(i_v, sh1)
        rseed = jnp.where(st | lane0m, p_v, -1)
        r = plsc.cummax(rseed)
        m, p = x_v, p_v
        for k in (1, 2, 4, 8):
            idxk = jnp.maximum(iota - k, 0)
            m_sh = _take(m, idxk)
            p_sh = _take(p, idxk)
            same = (jnp.logical_not(st) if k == 1
                    else _take(i_v, idxk) == i_v)
            tk = same & (m_sh >= m)
            m = jnp.where(tk, m_sh, m)
            p = jnp.where(tk, p_sh, p)
        fr = i_v == cs
        tc = fr & (cm >= m)
        m = jnp.where(tc, cm, m)
        p = jnp.where(tc, cp, p)
        r = jnp.where(fr, cr, r)
        plsc.store_scatter(ovm, [i_v], p - r, mask=r >= 0)
        return (_take(i_v, lane15), _take(m, lane15), _take(p, lane15),
                _take(r, lane15))

    def outer_body(b, carry):
        pltpu.sync_copy(x_hbm.at[pl.ds(al(base + b * blk), blk)], xb)
        pltpu.sync_copy(i_hbm.at[pl.ds(al(base + b * blk), blk)], ib)

        def block_body(t, carry):
            cs, cm, cp, cr = carry
            for u in range(5):
                off = t * 80 + u * 16
                x_v = xb[pl.ds(off, 16)]
                i_v = ib[pl.ds(off, 16)]
                cs, cm, cp, cr = step(
                    x_v, i_v, base + b * blk + off, cs, cm, cp, cr)
            return cs, cm, cp, cr
        return lax.fori_loop(0, blk // 80, block_body, carry)

    cs, cm, cp, cr = lax.fori_loop(0, c // blk, outer_body, (cs, cm, cp, cr))

    cs_s = jnp.max(cs)
    cr_s = jnp.max(cr)
    cm_s = jnp.max(cm)
    cp_s = jnp.max(cp)
    pos0 = base + c
    owned = cr_s >= 0

    def tail_cond(state):
        pos, closed, _, _ = state
        return jnp.logical_not(closed) & (pos < n)

    def tail_body(state):
        pos, closed, cm_s, cp_s = state
        rpos = jnp.minimum(pos, n - tail)
        pltpu.sync_copy(x_hbm.at[pl.ds(al(rpos), tail)], xtb)
        pltpu.sync_copy(i_hbm.at[pl.ds(al(rpos), tail)], itb)
        for j in range(tail // 16):
            x_v = xtb[pl.ds(j * 16, 16)]
            i_v = itb[pl.ds(j * 16, 16)]
            p_v = rpos + j * 16 + iota
            mism = i_v != cs_s
            pm = (jnp.cumsum(mism.astype(jnp.int32)) == 0) & jnp.logical_not(closed)
            vm = jnp.max(jnp.where(pm, x_v, neg_inf))
            vp = jnp.min(jnp.where(pm & (x_v == vm), p_v, _IMAX))
            better = vm > cm_s
            nonempty = jnp.any(pm)
            cm_s = jnp.where(nonempty & better, vm, cm_s)
            cp_s = jnp.where(nonempty & better, vp, cp_s)
            closed = closed | jnp.any(mism)
        return rpos + tail, closed, cm_s, cp_s

    _, _, cm_s, cp_s = lax.while_loop(
        tail_cond, tail_body,
        (pos0, jnp.logical_not(owned) | (pos0 >= n), cm_s, cp_s))

    plsc.store_scatter(ovm, [jnp.broadcast_to(cs_s, (16,))],
                       jnp.broadcast_to(cp_s - cr_s, (16,)),
                       mask=lane0m & owned)

    nbo = (nf_hi - al_lo + 127) // 128

    def copy_body(b, _):
        o0 = al(al_lo + b * 128)
        for j in range(8):
            iv = o0 + j * 16 + iota
            iv = jnp.where((iv >= nf_lo) & (iv < nf_hi), iv, s)
            iidx[pl.ds(j * 16, 16)] = iv
        pltpu.sync_copy(ovm.at[pl.ds(o0, 128)], out_hbm.at[iidx])
        return 0
    lax.fori_loop(0, nbo, copy_body, 0)


def _make(n, s, c, blk, tail):
    mesh = plsc.VectorSubcoreMesh(
        core_axis_name="c", subcore_axis_name="s",
        num_cores=_NC, num_subcores=_NS)
    body = functools.partial(_body, n=n, s=s, c=c, blk=blk, tail=tail)
    return pl.kernel(
        body,
        out_type=jax.ShapeDtypeStruct((s + 128,), jnp.int32),
        mesh=mesh,
        compiler_params=pltpu.CompilerParams(needs_layout_passes=False),
        scratch_types=[
            pltpu.VMEM((blk,), jnp.float32),
            pltpu.VMEM((blk,), jnp.int32),
            pltpu.VMEM((tail,), jnp.float32),
            pltpu.VMEM((tail,), jnp.int32),
            pltpu.VMEM((s + 128,), jnp.int32),
            pltpu.VMEM((128,), jnp.int32),
            pltpu.VMEM((16,), jnp.int32),
            pltpu.VMEM((16,), jnp.int32),
            pltpu.VMEM((16,), jnp.int32),
            pltpu.VMEM((16,), jnp.int32),
        ],
    )


def kernel(x, index):
    out = _make(_N, _S, _N // (_NC * _NS), _BLK, _TAIL)(x, index)
    return out[:_S]
